# Initial kernel scaffold; baseline (speedup 1.0000x reference)
import jax
import jax.numpy as jnp
from jax.experimental import pallas as pl


def kernel(x, freqs_cis, w_qkv, w_o):
    raise NotImplementedError("write your pallas kernel here")


# speedup vs baseline: 1.0808x; 1.0808x over previous
import functools

import jax
import jax.numpy as jnp
from jax.experimental import pallas as pl
from jax.experimental.pallas import tpu as pltpu

B = 1
S = 2048
HID = 2048
NH = 16
NKV = 8
HD = 128
WINDOW = 1024
SOFTCAP = 50.0
SCALING = HD ** -0.5

BM_P = 256
BN_P = 512
N_QKV = (NH + 2 * NKV) * HD
N_ROT_BLOCKS = (NH + NKV) * HD // BN_P


def _qkv_kernel(x_ref, w_ref, cos_ref, sin_ref, o_ref):
    j = pl.program_id(1)
    y = jax.lax.dot_general(
        x_ref[...], w_ref[...], (((1,), (1,)), ((), ())),
        preferred_element_type=jnp.float32)
    yh = y.reshape(BM_P, BN_P // HD, HD)
    rot_half = jnp.concatenate(
        [-yh[..., HD // 2:], yh[..., :HD // 2]], axis=-1)
    cos = cos_ref[...][:, None, :]
    sin = sin_ref[...][:, None, :]
    rot = (yh * cos + rot_half * sin).reshape(BM_P, BN_P)
    o_ref[...] = jnp.where(j < N_ROT_BLOCKS, rot, y)


def _qkv_proj(x2d, w_qkv, cos_full, sin_full):
    return pl.pallas_call(
        _qkv_kernel,
        grid=(S // BM_P, N_QKV // BN_P),
        in_specs=[
            pl.BlockSpec((BM_P, HID), lambda i, j: (i, 0)),
            pl.BlockSpec((BN_P, HID), lambda i, j: (j, 0)),
            pl.BlockSpec((BM_P, HD), lambda i, j: (i, 0)),
            pl.BlockSpec((BM_P, HD), lambda i, j: (i, 0)),
        ],
        out_specs=pl.BlockSpec((BM_P, BN_P), lambda i, j: (i, j)),
        out_shape=jax.ShapeDtypeStruct((S, N_QKV), jnp.float32),
    )(x2d, w_qkv, cos_full, sin_full)


BQ = 256
BK = 256
NT = WINDOW // BK + 1
Q_COLS = NH * HD


def _attn_kernel(q_ref, k_ref, v_ref, o_ref, denom_ref):
    qb = pl.program_id(1)
    t = pl.program_id(2)

    @pl.when(t == 0)
    def _init():
        o_ref[...] = jnp.zeros_like(o_ref)
        denom_ref[...] = jnp.zeros_like(denom_ref)

    @pl.when(qb + t - (NT - 1) >= 0)
    def _compute():
        q = q_ref[...]
        k = k_ref[...]
        v = v_ref[...]
        s = jax.lax.dot_general(
            q, k, (((1,), (1,)), ((), ())),
            preferred_element_type=jnp.float32) * SCALING
        s = jnp.tanh(s / SOFTCAP) * SOFTCAP
        p = jnp.exp(s)
        ri = jax.lax.broadcasted_iota(jnp.int32, (BQ, BK), 0)
        ci = jax.lax.broadcasted_iota(jnp.int32, (BQ, BK), 1)
        i = qb * BQ + ri
        j = (qb + t - (NT - 1)) * BK + ci
        keep = (j <= i) & (j > i - WINDOW)
        p = jnp.where(keep, p, 0.0)
        denom_ref[...] += jnp.sum(p, axis=1, keepdims=True)
        o_ref[...] += jax.lax.dot_general(
            p, v, (((1,), (0,)), ((), ())),
            preferred_element_type=jnp.float32)

    @pl.when(t == NT - 1)
    def _finish():
        o_ref[...] = o_ref[...] / denom_ref[...]


def _attention(qkv):
    def q_map(h, qb, t):
        return (qb, h)

    def kv_map(base):
        def f(h, qb, t):
            kb = jnp.maximum(qb + t - (NT - 1), 0)
            return (kb, base + h // (NH // NKV))
        return f

    return pl.pallas_call(
        _attn_kernel,
        grid=(NH, S // BQ, NT),
        in_specs=[
            pl.BlockSpec((BQ, HD), q_map),
            pl.BlockSpec((BK, HD), kv_map(NH)),
            pl.BlockSpec((BK, HD), kv_map(NH + NKV)),
        ],
        out_specs=pl.BlockSpec((BQ, HD), lambda h, qb, t: (qb, h)),
        out_shape=jax.ShapeDtypeStruct((S, NH * HD), jnp.float32),
        scratch_shapes=[pltpu.VMEM((BQ, 1), jnp.float32)],
    )(qkv, qkv, qkv)


BM_O = 256
BN_O = 512


def _oproj_kernel(a_ref, w_ref, o_ref):
    o_ref[...] = jax.lax.dot_general(
        a_ref[...], w_ref[...], (((1,), (1,)), ((), ())),
        preferred_element_type=jnp.float32)


def _out_proj(attn, w_o):
    return pl.pallas_call(
        _oproj_kernel,
        grid=(S // BM_O, HID // BN_O),
        in_specs=[
            pl.BlockSpec((BM_O, NH * HD), lambda i, j: (i, 0)),
            pl.BlockSpec((BN_O, NH * HD), lambda i, j: (j, 0)),
        ],
        out_specs=pl.BlockSpec((BM_O, BN_O), lambda i, j: (i, j)),
        out_shape=jax.ShapeDtypeStruct((S, HID), jnp.float32),
    )(attn, w_o)


@jax.jit
def kernel(x, freqs_cis, w_qkv, w_o):
    x2d = x.reshape(S, HID)
    cos = freqs_cis[..., 0]
    sin = freqs_cis[..., 1]
    cos_full = jnp.concatenate([cos, cos], axis=-1)
    sin_full = jnp.concatenate([sin, sin], axis=-1)
    qkv = _qkv_proj(x2d, w_qkv, cos_full, sin_full)
    attn = _attention(qkv)
    out = _out_proj(attn, w_o)
    return out.reshape(B, S, HID)


# speedup vs baseline: 2.8849x; 2.6693x over previous
import jax
import jax.numpy as jnp
from jax.experimental import pallas as pl
from jax.experimental.pallas import tpu as pltpu

B = 1
S = 2048
HID = 2048
NH = 16
NKV = 8
HD = 128
WINDOW = 1024
SOFTCAP = 50.0
SCALING = HD ** -0.5
LOG2E = 1.4426950408889634

BN_P = 512
N_QKV = (NH + 2 * NKV) * HD
N_ROT_BLOCKS = (NH + NKV) * HD // BN_P
Q_BLOCKS = NH * HD // BN_P


def _qkv_kernel(x_ref, w_ref, cos_ref, sin_ref, o_ref):
    j = pl.program_id(0)
    y = jax.lax.dot_general(
        x_ref[...].astype(jnp.bfloat16), w_ref[...].astype(jnp.bfloat16),
        (((1,), (1,)), ((), ())),
        preferred_element_type=jnp.float32)

    @pl.when(j < N_ROT_BLOCKS)
    def _rot():
        scale = jnp.where(j < Q_BLOCKS, SCALING / SOFTCAP, 1.0)
        yh = y.reshape(S, BN_P // HD, HD)
        rot_half = jnp.concatenate(
            [-yh[..., HD // 2:], yh[..., :HD // 2]], axis=-1)
        cos = cos_ref[...][:, None, :] * scale
        sin = sin_ref[...][:, None, :] * scale
        o_ref[...] = (yh * cos + rot_half * sin).reshape(
            S, BN_P).astype(jnp.bfloat16)

    @pl.when(j >= N_ROT_BLOCKS)
    def _plain():
        o_ref[...] = y.astype(jnp.bfloat16)


def _qkv_proj(x2d, w_qkv, cos_full, sin_full):
    return pl.pallas_call(
        _qkv_kernel,
        grid=(N_QKV // BN_P,),
        in_specs=[
            pl.BlockSpec((S, HID), lambda j: (0, 0)),
            pl.BlockSpec((BN_P, HID), lambda j: (j, 0)),
            pl.BlockSpec((S, HD), lambda j: (0, 0)),
            pl.BlockSpec((S, HD), lambda j: (0, 0)),
        ],
        out_specs=pl.BlockSpec((S, BN_P), lambda j: (0, j)),
        out_shape=jax.ShapeDtypeStruct((S, N_QKV), jnp.bfloat16),
    )(x2d, w_qkv, cos_full, sin_full)


BQ = 512
BK = 512
NT = WINDOW // BK + 1
C2 = SOFTCAP * LOG2E
NREP = NH // NKV


def _attn_kernel(q_ref, k_ref, v_ref, o_ref, denom_ref, acc_ref, mask_ref):
    g = pl.program_id(0)
    qb = pl.program_id(1)
    t = pl.program_id(2)

    @pl.when((g == 0) & (qb == 0) & (t == 0))
    def _build_masks():
        ri = jax.lax.broadcasted_iota(jnp.int32, (BQ, BK), 0)
        ci = jax.lax.broadcasted_iota(jnp.int32, (BQ, BK), 1)
        mask_ref[0] = (ci > ri).astype(jnp.float32)
        for mid in range(1, NT - 1):
            mask_ref[mid] = jnp.ones((BQ, BK), jnp.float32)
        mask_ref[NT - 1] = (ci <= ri).astype(jnp.float32)

    @pl.when(t == 0)
    def _init():
        denom_ref[...] = jnp.zeros_like(denom_ref)
        acc_ref[...] = jnp.zeros_like(acc_ref)

    @pl.when(qb + t - (NT - 1) >= 0)
    def _compute():
        k = k_ref[...]
        v = v_ref[...]
        m = mask_ref[t]
        for s in range(NREP):
            u = jax.lax.dot_general(
                q_ref[:, s * HD:(s + 1) * HD], k, (((1,), (1,)), ((), ())),
                preferred_element_type=jnp.float32)
            p = jnp.exp2(jnp.tanh(u) * C2) * m
            denom_ref[:, s:s + 1] += jnp.sum(p, axis=1, keepdims=True)
            acc_ref[:, s * HD:(s + 1) * HD] += jax.lax.dot_general(
                p.astype(jnp.bfloat16), v, (((1,), (0,)), ((), ())),
                preferred_element_type=jnp.float32)

    @pl.when(t == NT - 1)
    def _finish():
        for s in range(NREP):
            o_ref[:, s * HD:(s + 1) * HD] = (
                acc_ref[:, s * HD:(s + 1) * HD] / denom_ref[:, s:s + 1]
            ).astype(jnp.bfloat16)


def _attention(qkv):
    return pl.pallas_call(
        _attn_kernel,
        grid=(NKV, S // BQ, NT),
        in_specs=[
            pl.BlockSpec((BQ, NREP * HD), lambda g, qb, t: (qb, g)),
            pl.BlockSpec((BK, HD), lambda g, qb, t:
                         (jnp.maximum(qb + t - (NT - 1), 0), NH + g)),
            pl.BlockSpec((BK, HD), lambda g, qb, t:
                         (jnp.maximum(qb + t - (NT - 1), 0), NH + NKV + g)),
        ],
        out_specs=pl.BlockSpec((BQ, NREP * HD), lambda g, qb, t: (qb, g)),
        out_shape=jax.ShapeDtypeStruct((S, NH * HD), jnp.bfloat16),
        scratch_shapes=[pltpu.VMEM((BQ, NREP), jnp.float32),
                        pltpu.VMEM((BQ, NREP * HD), jnp.float32),
                        pltpu.VMEM((NT, BQ, BK), jnp.float32)],
    )(qkv, qkv, qkv)


BM_O = 512


def _oproj_kernel(a_ref, w_ref, o_ref):
    o_ref[...] = jax.lax.dot_general(
        a_ref[...], w_ref[...].astype(jnp.bfloat16),
        (((1,), (1,)), ((), ())),
        preferred_element_type=jnp.float32)


def _out_proj(attn, w_o):
    return pl.pallas_call(
        _oproj_kernel,
        grid=(S // BM_O,),
        in_specs=[
            pl.BlockSpec((BM_O, NH * HD), lambda i: (i, 0)),
            pl.BlockSpec((HID, NH * HD), lambda i: (0, 0)),
        ],
        out_specs=pl.BlockSpec((BM_O, HID), lambda i: (i, 0)),
        out_shape=jax.ShapeDtypeStruct((S, HID), jnp.float32),
    )(attn, w_o)


@jax.jit
def kernel(x, freqs_cis, w_qkv, w_o):
    x2d = x.reshape(S, HID)
    cos = freqs_cis[..., 0]
    sin = freqs_cis[..., 1]
    cos_full = jnp.concatenate([cos, cos], axis=-1)
    sin_full = jnp.concatenate([sin, sin], axis=-1)
    qkv = _qkv_proj(x2d, w_qkv, cos_full, sin_full)
    attn = _attention(qkv)
    out = _out_proj(attn, w_o)
    return out.reshape(B, S, HID)
